# baseline (speedup 1.0000x reference)
import functools

import jax
import jax.numpy as jnp
from jax import lax
from jax.experimental import pallas as pl
from jax.experimental.pallas import tpu as pltpu
from jax.experimental.pallas import tpu_sc as plsc

N_FEATURES = 26
NUM_EMBEDDINGS = 100000
EMBEDDING_DIM = 16
BATCH = 16384

TOTAL = BATCH * N_FEATURES
NUM_WORKERS = 32
PER_WORKER = TOTAL // NUM_WORKERS
IDX_MINOR = 128
ROWS_PER_WORKER = PER_WORKER // IDX_MINOR
BUF_ROWS = 1024
DMAS_PER_BUF = BUF_ROWS // IDX_MINOR
NUM_BUFS = PER_WORKER // BUF_ROWS

_mesh = plsc.VectorSubcoreMesh(core_axis_name="c", subcore_axis_name="s")


@functools.partial(
    pl.kernel,
    mesh=_mesh,
    out_type=jax.ShapeDtypeStruct((TOTAL, EMBEDDING_DIM), jnp.float32),
    scratch_types=[
        pltpu.VMEM((ROWS_PER_WORKER, IDX_MINOR), jnp.int32),
        pltpu.VMEM((BUF_ROWS, EMBEDDING_DIM), jnp.float32),
        pltpu.SemaphoreType.DMA,
    ],
    compiler_params=pltpu.CompilerParams(use_tc_tiling_on_sc=False),
)
def _gather_all(idx_hbm, table_hbm, out_hbm, idx_v, rows_v, sem):
    wid = lax.axis_index("s") * 2 + lax.axis_index("c")
    idx_row0 = wid * ROWS_PER_WORKER
    out0 = wid * PER_WORKER

    pltpu.sync_copy(idx_hbm.at[pl.ds(idx_row0, ROWS_PER_WORKER), :], idx_v)

    def buf_step(t, _):
        for j in range(DMAS_PER_BUF):
            pltpu.async_copy(
                table_hbm.at[idx_v.at[t * DMAS_PER_BUF + j]],
                rows_v.at[pl.ds(j * IDX_MINOR, IDX_MINOR), :],
                sem,
            ).wait()
        pltpu.sync_copy(
            rows_v, out_hbm.at[pl.ds(out0 + t * BUF_ROWS, BUF_ROWS), :]
        )
        return ()

    lax.fori_loop(0, NUM_BUFS, buf_step, ())


def kernel(x, embedding_weights):
    table = embedding_weights.reshape(N_FEATURES * NUM_EMBEDDINGS, EMBEDDING_DIM)
    offsets = (jnp.arange(N_FEATURES, dtype=jnp.int32) * NUM_EMBEDDINGS)[None, :]
    idx = (x.astype(jnp.int32) + offsets).reshape(TOTAL // IDX_MINOR, IDX_MINOR)
    out = _gather_all(idx, table)
    return out.reshape(BATCH, N_FEATURES, EMBEDDING_DIM)


# speedup vs baseline: 1.0402x; 1.0402x over previous
import functools

import jax
import jax.numpy as jnp
from jax import lax
from jax.experimental import pallas as pl
from jax.experimental.pallas import tpu as pltpu
from jax.experimental.pallas import tpu_sc as plsc

N_FEATURES = 26
NUM_EMBEDDINGS = 100000
EMBEDDING_DIM = 16
BATCH = 16384

TOTAL = BATCH * N_FEATURES
NUM_WORKERS = 32
PER_WORKER = TOTAL // NUM_WORKERS
BUF_ROWS = 512
NUM_BUFS = PER_WORKER // BUF_ROWS
NSLOT = 4

_mesh = plsc.VectorSubcoreMesh(core_axis_name="c", subcore_axis_name="s")


@functools.partial(
    pl.kernel,
    mesh=_mesh,
    out_type=jax.ShapeDtypeStruct((TOTAL, EMBEDDING_DIM), jnp.float32),
    scratch_types=[
        pltpu.VMEM((PER_WORKER,), jnp.int32),
        pltpu.VMEM((NSLOT, BUF_ROWS, EMBEDDING_DIM), jnp.float32),
        pltpu.SemaphoreType.DMA,
        pltpu.SemaphoreType.DMA,
    ],
    compiler_params=pltpu.CompilerParams(use_tc_tiling_on_sc=False),
)
def _gather_all(idx_hbm, table_hbm, out_hbm, idx_v, rows_v, gsem, wsem):
    wid = lax.axis_index("s") * 2 + lax.axis_index("c")
    out0 = wid * PER_WORKER

    pltpu.sync_copy(idx_hbm.at[pl.ds(out0, PER_WORKER)], idx_v)

    def fire_gather(t, slot):
        pltpu.async_copy(
            table_hbm.at[idx_v.at[pl.ds(t * BUF_ROWS, BUF_ROWS)]],
            rows_v.at[slot],
            gsem,
        )

    fire_gather(0, 0)
    fire_gather(1, 1)

    def visit(t, slot):
        pltpu.make_async_copy(
            table_hbm.at[pl.ds(0, BUF_ROWS), :], rows_v.at[slot], gsem
        ).wait()
        pltpu.async_copy(
            rows_v.at[slot],
            out_hbm.at[pl.ds(out0 + t * BUF_ROWS, BUF_ROWS), :],
            wsem,
        )
        @pl.when(t >= 2)
        def _():
            pltpu.make_async_copy(
                rows_v.at[(slot + 2) % NSLOT],
                out_hbm.at[pl.ds(0, BUF_ROWS), :],
                wsem,
            ).wait()

        @pl.when(t + 2 < NUM_BUFS)
        def _():
            fire_gather(t + 2, (slot + 2) % NSLOT)

    def ring_step(g, _):
        for b in range(NSLOT):
            visit(g * NSLOT + b, b)
        return ()

    lax.fori_loop(0, NUM_BUFS // NSLOT, ring_step, ())

    for t in range(NUM_BUFS - NUM_BUFS % NSLOT, NUM_BUFS):
        visit(t, t % NSLOT)

    for _ in range(2):
        pltpu.make_async_copy(
            rows_v.at[0], out_hbm.at[pl.ds(0, BUF_ROWS), :], wsem
        ).wait()


def kernel(x, embedding_weights):
    table = embedding_weights.reshape(N_FEATURES * NUM_EMBEDDINGS, EMBEDDING_DIM)
    offsets = (jnp.arange(N_FEATURES, dtype=jnp.int32) * NUM_EMBEDDINGS)[None, :]
    idx = (x.astype(jnp.int32) + offsets).reshape(TOTAL)
    out = _gather_all(idx, table)
    return out.reshape(BATCH, N_FEATURES, EMBEDDING_DIM)


# speedup vs baseline: 1.2015x; 1.1551x over previous
import functools

import jax
import jax.numpy as jnp
from jax import lax
from jax.experimental import pallas as pl
from jax.experimental.pallas import tpu as pltpu
from jax.experimental.pallas import tpu_sc as plsc

N_FEATURES = 26
NUM_EMBEDDINGS = 100000
EMBEDDING_DIM = 16
BATCH = 16384

NUM_WORKERS = 32
NB = BATCH // NUM_WORKERS
NSLOT = 4
LANES = 16

_mesh = plsc.VectorSubcoreMesh(core_axis_name="c", subcore_axis_name="s")


@functools.partial(
    pl.kernel,
    mesh=_mesh,
    out_type=jax.ShapeDtypeStruct((BATCH, N_FEATURES, EMBEDDING_DIM), jnp.float32),
    scratch_types=[
        pltpu.VMEM((NB, N_FEATURES), jnp.int32),
        pltpu.VMEM((NSLOT, NB), jnp.int32),
        pltpu.VMEM((NSLOT, NB, EMBEDDING_DIM), jnp.float32),
        pltpu.SemaphoreType.DMA,
        pltpu.SemaphoreType.DMA,
    ],
    compiler_params=pltpu.CompilerParams(
        use_tc_tiling_on_sc=False, needs_layout_passes=False
    ),
)
def _gather_all(x_hbm, table_hbm, out_hbm, xb_v, idx_v, rows_v, gsem, wsem):
    wid = lax.axis_index("s") * 2 + lax.axis_index("c")
    b0 = wid * NB

    pltpu.sync_copy(x_hbm.at[pl.ds(b0, NB), :], xb_v)

    iota16 = jax.lax.iota(jnp.int32, LANES)

    def extract_idx(f):
        colf = jnp.full((LANES,), f, jnp.int32)
        slot = f % NSLOT

        def body(i, _):
            rows = i * LANES + iota16
            vals = plsc.load_gather(xb_v, [rows, colf])
            idx_v[slot, pl.ds(i * LANES, LANES)] = vals
            return ()

        lax.fori_loop(0, NB // LANES, body, ())

    def fire_gather(f):
        pltpu.async_copy(
            table_hbm.at[f].at[idx_v.at[f % NSLOT]],
            rows_v.at[f % NSLOT],
            gsem,
        )

    def drain_gather(f):
        pltpu.make_async_copy(
            table_hbm.at[0, pl.ds(0, NB), :], rows_v.at[f % NSLOT], gsem
        ).wait()

    def fire_write(f):
        pltpu.async_copy(
            rows_v.at[f % NSLOT],
            out_hbm.at[pl.ds(b0, NB), f],
            wsem,
        )

    def drain_write(f):
        pltpu.make_async_copy(
            rows_v.at[f % NSLOT], out_hbm.at[pl.ds(0, NB), 0], wsem
        ).wait()

    for f in range(2):
        extract_idx(f)
        fire_gather(f)

    for f in range(N_FEATURES):
        if f + 2 < N_FEATURES:
            extract_idx(f + 2)
        drain_gather(f)
        fire_write(f)
        if f >= 2:
            drain_write(f - 2)
        if f + 2 < N_FEATURES:
            fire_gather(f + 2)

    for f in range(N_FEATURES - 2, N_FEATURES):
        drain_write(f)


def kernel(x, embedding_weights):
    return _gather_all(x, embedding_weights)


# speedup vs baseline: 2.8337x; 2.3585x over previous
import functools

import jax
import jax.numpy as jnp
from jax import lax
from jax.experimental import pallas as pl
from jax.experimental.pallas import tpu as pltpu
from jax.experimental.pallas import tpu_sc as plsc

N_FEATURES = 26
NUM_EMBEDDINGS = 100000
EMBEDDING_DIM = 16
BATCH = 16384

NUM_WORKERS = 32
NB = BATCH // NUM_WORKERS
LANES = 16

_mesh = plsc.VectorSubcoreMesh(core_axis_name="c", subcore_axis_name="s")


@functools.partial(
    pl.kernel,
    mesh=_mesh,
    out_type=jax.ShapeDtypeStruct((N_FEATURES, EMBEDDING_DIM, BATCH), jnp.float32),
    scratch_types=[
        pltpu.VMEM((NB, N_FEATURES), jnp.int32),
        pltpu.VMEM((2, NB), jnp.int32),
        pltpu.VMEM((2, EMBEDDING_DIM, NB), jnp.float32),
        pltpu.SemaphoreType.DMA,
        pltpu.SemaphoreType.DMA,
    ],
    compiler_params=pltpu.CompilerParams(
        use_tc_tiling_on_sc=False, needs_layout_passes=False
    ),
)
def _gather_all(x_hbm, wt_hbm, out_hbm, xb_v, idx_v, fbuf_v, gsem, wsem):
    wid = lax.axis_index("s") * 2 + lax.axis_index("c")
    b0 = wid * NB

    pltpu.sync_copy(x_hbm.at[pl.ds(b0, NB), :], xb_v)

    iota16 = jax.lax.iota(jnp.int32, LANES)

    def extract_idx(f, slot):
        colf = jnp.full((LANES,), 0, jnp.int32) + f

        def body(i, _):
            rows = i * LANES + iota16
            vals = plsc.load_gather(xb_v, [rows, colf])
            idx_v[slot, pl.ds(i * LANES, LANES)] = vals
            return ()

        lax.fori_loop(0, NB // LANES, body, ())

    def fire_gathers(f, slot):
        for d in range(EMBEDDING_DIM):
            pltpu.async_copy(
                wt_hbm.at[f, d].at[idx_v.at[slot]],
                fbuf_v.at[slot, d],
                gsem,
            )

    def drain_gathers(slot):
        pltpu.make_async_copy(
            wt_hbm.at[0, pl.ds(0, EMBEDDING_DIM), pl.ds(0, NB)],
            fbuf_v.at[slot],
            gsem,
        ).wait()

    def fire_write(f, slot):
        pltpu.async_copy(
            fbuf_v.at[slot],
            out_hbm.at[f, :, pl.ds(b0, NB)],
            wsem,
        )

    def drain_write(slot):
        pltpu.make_async_copy(
            fbuf_v.at[slot], out_hbm.at[0, :, pl.ds(0, NB)], wsem
        ).wait()

    extract_idx(0, 0)

    def step(f, _):
        slot = lax.rem(f, 2)
        fire_gathers(f, slot)

        @pl.when(f + 1 < N_FEATURES)
        def _():
            extract_idx(f + 1, 1 - slot)

        @pl.when(f >= 1)
        def _():
            drain_write(1 - slot)

        drain_gathers(slot)
        fire_write(f, slot)
        return ()

    lax.fori_loop(0, N_FEATURES, step, ())
    drain_write((N_FEATURES - 1) % 2)


def kernel(x, embedding_weights):
    wt = jnp.transpose(embedding_weights, (0, 2, 1))
    out_t = _gather_all(x, wt)
    return jnp.transpose(out_t, (2, 0, 1))


# speedup vs baseline: 2.9391x; 1.0372x over previous
import functools

import jax
import jax.numpy as jnp
from jax import lax
from jax.experimental import pallas as pl
from jax.experimental.pallas import tpu as pltpu
from jax.experimental.pallas import tpu_sc as plsc

N_FEATURES = 26
NUM_EMBEDDINGS = 100000
EMBEDDING_DIM = 16
BATCH = 16384

NUM_WORKERS = 32
NB = BATCH // NUM_WORKERS
LANES = 16
CHUNKS = (7, 7, 6, 6)

_mesh = plsc.VectorSubcoreMesh(core_axis_name="c", subcore_axis_name="s")


def _make_chunk_kernel(nf, fbase):
    @functools.partial(
        pl.kernel,
        mesh=_mesh,
        out_type=jax.ShapeDtypeStruct((nf, EMBEDDING_DIM, BATCH), jnp.float32),
        scratch_types=[
            pltpu.VMEM((NB, N_FEATURES), jnp.int32),
            pltpu.VMEM((2, NB), jnp.int32),
            pltpu.VMEM((2, EMBEDDING_DIM, NB), jnp.float32),
            pltpu.SemaphoreType.DMA,
            pltpu.SemaphoreType.DMA,
        ],
        compiler_params=pltpu.CompilerParams(
            use_tc_tiling_on_sc=False, needs_layout_passes=False
        ),
    )
    def _gather_chunk(x_hbm, wt_hbm, out_hbm, xb_v, idx_v, fbuf_v, gsem, wsem):
        wid = lax.axis_index("s") * 2 + lax.axis_index("c")
        b0 = wid * NB

        pltpu.sync_copy(x_hbm.at[pl.ds(b0, NB), :], xb_v)

        iota16 = jax.lax.iota(jnp.int32, LANES)

        def extract_idx(f, slot):
            colf = jnp.full((LANES,), fbase, jnp.int32) + f

            def body(i, _):
                rows = i * LANES + iota16
                vals = plsc.load_gather(xb_v, [rows, colf])
                idx_v[slot, pl.ds(i * LANES, LANES)] = vals
                return ()

            lax.fori_loop(0, NB // LANES, body, ())

        def fire_gathers(f, slot):
            for d in range(EMBEDDING_DIM):
                pltpu.async_copy(
                    wt_hbm.at[f, d].at[idx_v.at[slot]],
                    fbuf_v.at[slot, d],
                    gsem,
                )

        def drain_gathers(slot):
            pltpu.make_async_copy(
                wt_hbm.at[0, pl.ds(0, EMBEDDING_DIM), pl.ds(0, NB)],
                fbuf_v.at[slot],
                gsem,
            ).wait()

        def fire_write(f, slot):
            pltpu.async_copy(
                fbuf_v.at[slot],
                out_hbm.at[f, :, pl.ds(b0, NB)],
                wsem,
            )

        def drain_write(slot):
            pltpu.make_async_copy(
                fbuf_v.at[slot], out_hbm.at[0, :, pl.ds(0, NB)], wsem
            ).wait()

        extract_idx(0, 0)

        def step(f, _):
            slot = lax.rem(f, 2)
            fire_gathers(f, slot)

            @pl.when(f + 1 < nf)
            def _():
                extract_idx(f + 1, 1 - slot)

            @pl.when(f >= 1)
            def _():
                drain_write(1 - slot)

            drain_gathers(slot)
            fire_write(f, slot)
            return ()

        lax.fori_loop(0, nf, step, ())
        drain_write((nf - 1) % 2)

    return _gather_chunk


_chunk_kernels = []
_base = 0
for _nf in CHUNKS:
    _chunk_kernels.append((_make_chunk_kernel(_nf, _base), _base, _nf))
    _base += _nf


def kernel(x, embedding_weights):
    wt = jnp.transpose(embedding_weights, (0, 2, 1))
    outs = []
    for fn, fbase, nf in _chunk_kernels:
        outs.append(fn(x, lax.slice_in_dim(wt, fbase, fbase + nf, axis=0)))
    out_t = jnp.concatenate(outs, axis=0)
    return jnp.transpose(out_t, (2, 0, 1))


# speedup vs baseline: 2.9668x; 1.0094x over previous
import functools

import jax
import jax.numpy as jnp
from jax import lax
from jax.experimental import pallas as pl
from jax.experimental.pallas import tpu as pltpu
from jax.experimental.pallas import tpu_sc as plsc

N_FEATURES = 26
NUM_EMBEDDINGS = 100000
EMBEDDING_DIM = 16
BATCH = 16384

NUM_WORKERS = 32
NB = BATCH // NUM_WORKERS
LANES = 16
CHUNKS = (7, 7, 6, 6)
NSLOT = 4

_mesh = plsc.VectorSubcoreMesh(core_axis_name="c", subcore_axis_name="s")


def _make_chunk_kernel(nf, fbase):
    @functools.partial(
        pl.kernel,
        mesh=_mesh,
        out_type=jax.ShapeDtypeStruct((nf, EMBEDDING_DIM, BATCH), jnp.float32),
        scratch_types=[
            pltpu.VMEM((NB, N_FEATURES), jnp.int32),
            pltpu.VMEM((NSLOT, NB), jnp.int32),
            pltpu.VMEM((NSLOT, EMBEDDING_DIM, NB), jnp.float32),
            pltpu.SemaphoreType.DMA,
            pltpu.SemaphoreType.DMA,
        ],
        compiler_params=pltpu.CompilerParams(
            use_tc_tiling_on_sc=False, needs_layout_passes=False
        ),
    )
    def _gather_chunk(x_hbm, wt_hbm, out_hbm, xb_v, idx_v, fbuf_v, gsem, wsem):
        wid = lax.axis_index("s") * 2 + lax.axis_index("c")
        b0 = wid * NB

        pltpu.sync_copy(x_hbm.at[pl.ds(b0, NB), :], xb_v)

        iota16 = jax.lax.iota(jnp.int32, LANES)

        def extract_idx(f, slot):
            colf = jnp.full((LANES,), fbase, jnp.int32) + f

            def body(i, _):
                rows = i * LANES + iota16
                vals = plsc.load_gather(xb_v, [rows, colf])
                idx_v[slot, pl.ds(i * LANES, LANES)] = vals
                return ()

            lax.fori_loop(0, NB // LANES, body, ())

        def fire_gathers(f, slot):
            for d in range(EMBEDDING_DIM):
                row0 = (f * EMBEDDING_DIM + d) * NUM_EMBEDDINGS
                pltpu.async_copy(
                    wt_hbm.at[pl.ds(row0, NUM_EMBEDDINGS)].at[idx_v.at[slot]],
                    fbuf_v.at[slot, d],
                    gsem,
                )

        def drain_gathers(slot):
            pltpu.make_async_copy(
                out_hbm.at[0, :, pl.ds(0, NB)], fbuf_v.at[slot], gsem
            ).wait()

        def fire_write(f, slot):
            pltpu.async_copy(
                fbuf_v.at[slot],
                out_hbm.at[f, :, pl.ds(b0, NB)],
                wsem,
            )

        def drain_write(slot):
            pltpu.make_async_copy(
                fbuf_v.at[slot], out_hbm.at[0, :, pl.ds(0, NB)], wsem
            ).wait()

        extract_idx(0, 0)
        extract_idx(1, 1)
        fire_gathers(0, 0)

        def step(f, _):
            slot = lax.rem(f, NSLOT)

            @pl.when(f >= 3)
            def _():
                drain_write(lax.rem(f + 1, NSLOT))

            @pl.when(f + 1 < nf)
            def _():
                fire_gathers(f + 1, lax.rem(f + 1, NSLOT))

            @pl.when(f + 2 < nf)
            def _():
                extract_idx(f + 2, lax.rem(f + 2, NSLOT))

            drain_gathers(slot)
            fire_write(f, slot)
            return ()

        lax.fori_loop(0, nf, step, ())
        for t in range(min(3, nf)):
            drain_write((nf - min(3, nf) + t) % NSLOT)

    return _gather_chunk


_chunk_kernels = []
_base = 0
for _nf in CHUNKS:
    _chunk_kernels.append((_make_chunk_kernel(_nf, _base), _base, _nf))
    _base += _nf


def kernel(x, embedding_weights):
    wt = jnp.transpose(embedding_weights, (0, 2, 1))
    outs = []
    for fn, fbase, nf in _chunk_kernels:
        wflat = lax.slice_in_dim(wt, fbase, fbase + nf, axis=0).reshape(-1)
        outs.append(fn(x, wflat))
    out_t = jnp.concatenate(outs, axis=0)
    return jnp.transpose(out_t, (2, 0, 1))
